# baseline (speedup 1.0000x reference)
import functools
import math

import jax
import jax.numpy as jnp
from jax import lax
from jax.experimental import pallas as pl
from jax.experimental.pallas import tpu as pltpu
from jax.experimental.pallas import tpu_sc as plsc

D_MODEL = 64
SCALE = math.sqrt(D_MODEL)

NC = 2
NS = 16
NW = NC * NS

IDXW = 128
CHUNK = 512
GPC = CHUNK // IDXW


def _emb_kernel(n_rows, table_hbm, idx_hbm, out_hbm, idx_v, rows_v, sem):
    b_per_w = n_rows // NW
    n_chunks = b_per_w // CHUNK
    wid = lax.axis_index("s") * NC + lax.axis_index("c")
    base = wid * b_per_w

    def chunk_body(g, _):
        off = base + g * CHUNK
        pltpu.sync_copy(idx_hbm.at[pl.ds(off, CHUNK)], idx_v)
        copies = []
        for j in range(GPC):
            copies.append(
                pltpu.async_copy(
                    table_hbm.at[idx_v.at[pl.ds(j * IDXW, IDXW)]],
                    rows_v.at[pl.ds(j * IDXW, IDXW)],
                    sem,
                )
            )
        for c in copies:
            c.wait()

        def scale_row(r, _):
            for j in range(D_MODEL // 16):
                sl = pl.ds(j * 16, 16)
                rows_v[r, sl] = rows_v[r, sl] * SCALE
            return 0

        lax.fori_loop(0, CHUNK, scale_row, 0, unroll=2)

        pltpu.sync_copy(rows_v, out_hbm.at[pl.ds(off, CHUNK)])
        return 0

    lax.fori_loop(0, n_chunks, chunk_body, 0)


@jax.jit
def kernel(x, table):
    n_rows = x.size
    idx1d = x.reshape(n_rows)
    mesh = plsc.VectorSubcoreMesh(core_axis_name="c", subcore_axis_name="s")
    out = pl.kernel(
        functools.partial(_emb_kernel, n_rows),
        out_type=jax.ShapeDtypeStruct((n_rows, D_MODEL), jnp.float32),
        mesh=mesh,
        scratch_types=[
            pltpu.VMEM((CHUNK,), jnp.int32),
            pltpu.VMEM((CHUNK, D_MODEL), jnp.float32),
            pltpu.SemaphoreType.DMA,
        ],
        compiler_params=pltpu.CompilerParams(use_tc_tiling_on_sc=False),
    )(table, idx1d)
    return out.reshape(*x.shape, D_MODEL)
